# baseline (speedup 1.0000x reference)
import functools

import jax
import jax.numpy as jnp
from jax import lax
from jax.experimental import pallas as pl
from jax.experimental.pallas import tpu as pltpu
from jax.experimental.pallas import tpu_sc as plsc

N = 10000
D = 128
E = 320000

NC = 2
NS = 16
C = 128
K = 80
KB = 5
SK = K // KB
EPT = K * C
E_PAD = NC * NS * EPT
NP = 10240
RPT = NP // NS


def _sc_agg_build(with_cnt: bool):
    mesh = plsc.VectorSubcoreMesh(core_axis_name="c", subcore_axis_name="s")
    if with_cnt:
        out_type = (jax.ShapeDtypeStruct((NC, NP, D), jnp.float32),
                    jax.ShapeDtypeStruct((NC, NP), jnp.float32))
    else:
        out_type = jax.ShapeDtypeStruct((NC, NP, D), jnp.float32)

    scratch = (
        pltpu.VMEM((SK, C), jnp.int32),
        pltpu.VMEM((SK, C), jnp.int32),
        pltpu.VMEM((C, D), jnp.float32),
        pltpu.VMEM((C, D), jnp.float32),
        pltpu.VMEM((C,), jnp.float32),
        pltpu.VMEM((RPT,), jnp.float32),
        pltpu.VMEM_SHARED((NP, D), jnp.float32),
        pltpu.VMEM_SHARED((NP,), jnp.float32),
        pltpu.SemaphoreType.DMA,
        pltpu.SemaphoreType.DMA,
        pltpu.SemaphoreType.DMA,
        pltpu.SemaphoreType.DMA,
    )

    def body(x_hbm, src_hbm, dst_hbm, agg_out, *rest):
        if with_cnt:
            cnt_out = rest[0]
            rest = rest[1:]
        else:
            cnt_out = None
        (src_v, dst_v, rows0, rows1, ones_v, cntbuf, acc_sh, cnt_sh,
         gsem0, gsem1, ssem0, ssem1) = rest
        cid = lax.axis_index("c")
        sid = lax.axis_index("s")
        rows = (rows0, rows1)
        gsems = (gsem0, gsem1)
        ssems = (ssem0, ssem1)

        z16 = jnp.zeros((16,), jnp.float32)

        def zrow(i, _):
            for k in range(D // 16):
                rows0[i, pl.ds(k * 16, 16)] = z16
            return 0

        lax.fori_loop(0, C, zrow, 0)

        def zcnt(i, _):
            cntbuf[pl.ds(i * 16, 16)] = z16
            return 0

        lax.fori_loop(0, RPT // 16, zcnt, 0)
        for k in range(C // 16):
            ones_v[pl.ds(k * 16, 16)] = jnp.ones((16,), jnp.float32)

        base = pl.multiple_of(sid * RPT, RPT)
        for t in range(RPT // C):
            pltpu.sync_copy(rows0, acc_sh.at[pl.ds(base + t * C, C)])
        if with_cnt:
            pltpu.sync_copy(cntbuf, cnt_sh.at[pl.ds(base, RPT)])
        plsc.subcore_barrier()

        def gather(j, b):
            return pltpu.async_copy(x_hbm.at[src_v.at[j]], rows[b], gsems[b])

        def gwait(j, b):
            pltpu.make_async_copy(x_hbm.at[src_v.at[j]], rows[b], gsems[b]).wait()

        def sstart(j, b):
            pltpu.async_copy(rows[b], acc_sh.at[dst_v.at[j]], ssems[b], add=True)
            if with_cnt:
                pltpu.sync_copy(ones_v, cnt_sh.at[dst_v.at[j]], add=True)

        def swait(j, b):
            pltpu.make_async_copy(rows[b], acc_sh.at[dst_v.at[j]], ssems[b]).wait()

        def stage(st, _):
            pltpu.sync_copy(src_hbm.at[cid, sid, pl.ds(st * SK, SK)], src_v)
            pltpu.sync_copy(dst_hbm.at[cid, sid, pl.ds(st * SK, SK)], dst_v)
            gather(0, 0)
            gather(1, 1)

            def step(g, _):
                j = 2 * g
                for b in range(2):
                    gwait(j + b, b)
                    sstart(j + b, b)
                for b in range(2):
                    swait(j + b, b)
                    gather(j + b + 2, b)
                return 0

            lax.fori_loop(0, SK // 2 - 1, step, 0)
            for b in range(2):
                j = SK - 2 + b
                gwait(j, b)
                sstart(j, b)
            for b in range(2):
                swait(SK - 2 + b, b)
            return 0

        lax.fori_loop(0, KB, stage, 0)

        plsc.subcore_barrier()
        for t in range(RPT // C):
            pltpu.sync_copy(acc_sh.at[pl.ds(base + t * C, C)], rows0)
            pltpu.sync_copy(rows0, agg_out.at[cid, pl.ds(base + t * C, C)])
        if with_cnt:
            pltpu.sync_copy(cnt_sh.at[pl.ds(base, RPT)], cntbuf)
            pltpu.sync_copy(cntbuf, cnt_out.at[cid, pl.ds(base, RPT)])

    return pl.kernel(body, out_type=out_type, mesh=mesh,
                     scratch_types=scratch)


_sc_agg_cnt = _sc_agg_build(with_cnt=True)
_sc_agg = _sc_agg_build(with_cnt=False)


_RB = 1000
_dn = (((1,), (1,)), ((), ()))


def _tc_root_body(x_ref, wr_ref, b_ref, r_ref):
    r_ref[...] = lax.dot_general(
        x_ref[...], wr_ref[...], _dn, preferred_element_type=jnp.float32
    ) + b_ref[...]


_tc_root = pl.pallas_call(
    _tc_root_body,
    grid=(N // _RB,),
    in_specs=[
        pl.BlockSpec((_RB, D), lambda i: (i, 0)),
        pl.BlockSpec((D, D), lambda i: (0, 0)),
        pl.BlockSpec((1, D), lambda i: (0, 0)),
    ],
    out_specs=pl.BlockSpec((_RB, D), lambda i: (i, 0)),
    out_shape=jax.ShapeDtypeStruct((N, D), jnp.float32),
)


def _tc_combine_body(agg_ref, cnt_ref, r_ref, wl_ref, out_ref):
    cnt = cnt_ref[...]
    inv = 1.0 / jnp.maximum(cnt[:, 0:1] + cnt[:, 1:2], 1.0)
    mean = (agg_ref[0] + agg_ref[1]) * inv
    out_ref[...] = lax.dot_general(
        mean, wl_ref[...], _dn, preferred_element_type=jnp.float32
    ) + r_ref[...]


_tc_combine = pl.pallas_call(
    _tc_combine_body,
    grid=(N // _RB,),
    in_specs=[
        pl.BlockSpec((NC, _RB, D), lambda i: (0, i, 0)),
        pl.BlockSpec((_RB, NC), lambda i: (i, 0)),
        pl.BlockSpec((_RB, D), lambda i: (i, 0)),
        pl.BlockSpec((D, D), lambda i: (0, 0)),
    ],
    out_specs=pl.BlockSpec((_RB, D), lambda i: (i, 0)),
    out_shape=jax.ShapeDtypeStruct((N, D), jnp.float32),
)


def kernel(x, edge_index, x_batch, W1l, b1, W1r, W2l, b2, W2r):
    src = edge_index[0].astype(jnp.int32)
    dst = edge_index[1].astype(jnp.int32)
    npad = E_PAD - E
    src = jnp.concatenate([src, jnp.arange(npad, dtype=jnp.int32) % N])
    dst = jnp.concatenate([dst, N + (jnp.arange(npad, dtype=jnp.int32) % (NP - N))])
    src_r = src.reshape(NC, NS, K, C)
    dst_r = dst.reshape(NC, NS, K, C)

    agg1, cnt = _sc_agg_cnt(x, src_r, dst_r)
    cnt_t = cnt.T
    r1 = _tc_root(x, W1r, b1.reshape(1, D))
    h = _tc_combine(agg1, cnt_t, r1, W1l)
    agg2 = _sc_agg(h, src_r, dst_r)
    r2 = _tc_root(h, W2r, b2.reshape(1, D))
    out = _tc_combine(agg2, cnt_t, r2, W2l)
    return out


# speedup vs baseline: 1.2198x; 1.2198x over previous
import functools

import jax
import jax.numpy as jnp
from jax import lax
from jax.experimental import pallas as pl
from jax.experimental.pallas import tpu as pltpu
from jax.experimental.pallas import tpu_sc as plsc

N = 10000
D = 128
E = 320000

NC = 2
NS = 16
C = 128
K = 80
KB = 5
SK = K // KB
EPT = K * C
E_PAD = NC * NS * EPT
NP = 10240
RPT = NP // NS


def _sc_agg_build(with_cnt: bool):
    mesh = plsc.VectorSubcoreMesh(core_axis_name="c", subcore_axis_name="s")
    if with_cnt:
        out_type = (jax.ShapeDtypeStruct((NC, NP, D), jnp.float32),
                    jax.ShapeDtypeStruct((NC, NP), jnp.float32))
    else:
        out_type = jax.ShapeDtypeStruct((NC, NP, D), jnp.float32)

    scratch = (
        pltpu.VMEM((SK, C), jnp.int32),
        pltpu.VMEM((SK, C), jnp.int32),
        pltpu.VMEM((C, D), jnp.float32),
        pltpu.VMEM((C, D), jnp.float32),
        pltpu.VMEM((C,), jnp.float32),
        pltpu.VMEM((RPT,), jnp.float32),
        pltpu.VMEM_SHARED((NP, D), jnp.float32),
        pltpu.VMEM_SHARED((NP,), jnp.float32),
        pltpu.SemaphoreType.DMA,
        pltpu.SemaphoreType.DMA,
    )

    def body(x_hbm, src_hbm, dst_hbm, agg_out, *rest):
        if with_cnt:
            cnt_out = rest[0]
            rest = rest[1:]
        else:
            cnt_out = None
        (src_v, dst_v, rows0, rows1, ones_v, cntbuf, acc_sh, cnt_sh,
         gsem0, gsem1) = rest
        cid = lax.axis_index("c")
        sid = lax.axis_index("s")
        rows = (rows0, rows1)
        gsems = (gsem0, gsem1)

        z16 = jnp.zeros((16,), jnp.float32)

        def zrow(i, _):
            for k in range(D // 16):
                rows0[i, pl.ds(k * 16, 16)] = z16
            return 0

        lax.fori_loop(0, C, zrow, 0)

        def zcnt(i, _):
            cntbuf[pl.ds(i * 16, 16)] = z16
            return 0

        lax.fori_loop(0, RPT // 16, zcnt, 0)
        for k in range(C // 16):
            ones_v[pl.ds(k * 16, 16)] = jnp.ones((16,), jnp.float32)

        base = pl.multiple_of(sid * RPT, RPT)
        for t in range(RPT // C):
            pltpu.sync_copy(rows0, acc_sh.at[pl.ds(base + t * C, C)])
        if with_cnt:
            pltpu.sync_copy(cntbuf, cnt_sh.at[pl.ds(base, RPT)])
        plsc.subcore_barrier()

        def gather(j, b):
            return pltpu.async_copy(x_hbm.at[src_v.at[j]], rows[b], gsems[b])

        def gwait(j, b):
            pltpu.make_async_copy(x_hbm.at[src_v.at[j]], rows[b], gsems[b]).wait()

        def scat(j, b):
            pltpu.sync_copy(rows[b], acc_sh.at[dst_v.at[j]], add=True)
            if with_cnt:
                pltpu.sync_copy(ones_v, cnt_sh.at[dst_v.at[j]], add=True)

        def stage(st, _):
            pltpu.sync_copy(src_hbm.at[cid, sid, pl.ds(st * SK, SK)], src_v)
            pltpu.sync_copy(dst_hbm.at[cid, sid, pl.ds(st * SK, SK)], dst_v)
            gather(0, 0)
            gather(1, 1)

            def step(g, _):
                j = 2 * g
                for b in range(2):
                    gwait(j + b, b)
                    scat(j + b, b)
                    gather(j + b + 2, b)
                return 0

            lax.fori_loop(0, SK // 2 - 1, step, 0)
            for b in range(2):
                j = SK - 2 + b
                gwait(j, b)
                scat(j, b)
            return 0

        lax.fori_loop(0, KB, stage, 0)

        plsc.subcore_barrier()
        for t in range(RPT // C):
            pltpu.sync_copy(acc_sh.at[pl.ds(base + t * C, C)], rows0)
            pltpu.sync_copy(rows0, agg_out.at[cid, pl.ds(base + t * C, C)])
        if with_cnt:
            pltpu.sync_copy(cnt_sh.at[pl.ds(base, RPT)], cntbuf)
            pltpu.sync_copy(cntbuf, cnt_out.at[cid, pl.ds(base, RPT)])

    return pl.kernel(body, out_type=out_type, mesh=mesh,
                     scratch_types=scratch)


_sc_agg_cnt = _sc_agg_build(with_cnt=True)
_sc_agg = _sc_agg_build(with_cnt=False)


_RB = 1000
_dn = (((1,), (1,)), ((), ()))


def _tc_root_body(x_ref, wr_ref, b_ref, r_ref):
    r_ref[...] = lax.dot_general(
        x_ref[...], wr_ref[...], _dn, preferred_element_type=jnp.float32
    ) + b_ref[...]


_tc_root = pl.pallas_call(
    _tc_root_body,
    grid=(N // _RB,),
    in_specs=[
        pl.BlockSpec((_RB, D), lambda i: (i, 0)),
        pl.BlockSpec((D, D), lambda i: (0, 0)),
        pl.BlockSpec((1, D), lambda i: (0, 0)),
    ],
    out_specs=pl.BlockSpec((_RB, D), lambda i: (i, 0)),
    out_shape=jax.ShapeDtypeStruct((N, D), jnp.float32),
)


def _tc_combine_body(agg_ref, cnt_ref, r_ref, wl_ref, out_ref):
    cnt = cnt_ref[...]
    inv = 1.0 / jnp.maximum(cnt[:, 0:1] + cnt[:, 1:2], 1.0)
    mean = (agg_ref[0] + agg_ref[1]) * inv
    out_ref[...] = lax.dot_general(
        mean, wl_ref[...], _dn, preferred_element_type=jnp.float32
    ) + r_ref[...]


_tc_combine = pl.pallas_call(
    _tc_combine_body,
    grid=(N // _RB,),
    in_specs=[
        pl.BlockSpec((NC, _RB, D), lambda i: (0, i, 0)),
        pl.BlockSpec((_RB, NC), lambda i: (i, 0)),
        pl.BlockSpec((_RB, D), lambda i: (i, 0)),
        pl.BlockSpec((D, D), lambda i: (0, 0)),
    ],
    out_specs=pl.BlockSpec((_RB, D), lambda i: (i, 0)),
    out_shape=jax.ShapeDtypeStruct((N, D), jnp.float32),
)


def kernel(x, edge_index, x_batch, W1l, b1, W1r, W2l, b2, W2r):
    src = edge_index[0].astype(jnp.int32)
    dst = edge_index[1].astype(jnp.int32)
    npad = E_PAD - E
    src = jnp.concatenate([src, jnp.arange(npad, dtype=jnp.int32) % N])
    dst = jnp.concatenate([dst, N + (jnp.arange(npad, dtype=jnp.int32) % (NP - N))])
    src_r = src.reshape(NC, NS, K, C)
    dst_r = dst.reshape(NC, NS, K, C)

    agg1, cnt = _sc_agg_cnt(x, src_r, dst_r)
    cnt_t = cnt.T
    r1 = _tc_root(x, W1r, b1.reshape(1, D))
    h = _tc_combine(agg1, cnt_t, r1, W1l)
    agg2 = _sc_agg(h, src_r, dst_r)
    r2 = _tc_root(h, W2r, b2.reshape(1, D))
    out = _tc_combine(agg2, cnt_t, r2, W2l)
    return out


# speedup vs baseline: 1.2478x; 1.0229x over previous
import functools

import jax
import jax.numpy as jnp
from jax import lax
from jax.experimental import pallas as pl
from jax.experimental.pallas import tpu as pltpu
from jax.experimental.pallas import tpu_sc as plsc

N = 10000
D = 128
DH = D // 2
E = 320000

NC = 2
NS = 16
C = 128
K = 160
NBUF = 4
EPT = K * C
E_PAD = NS * EPT
NP = 10240
RPT = NP // NS


def _sc_agg_build(with_cnt: bool):
    mesh = plsc.VectorSubcoreMesh(core_axis_name="c", subcore_axis_name="s")
    if with_cnt:
        out_type = (jax.ShapeDtypeStruct((NC, NP, DH), jnp.float32),
                    jax.ShapeDtypeStruct((NC, NP), jnp.float32))
    else:
        out_type = jax.ShapeDtypeStruct((NC, NP, DH), jnp.float32)

    scratch = (
        pltpu.VMEM((K, C), jnp.int32),
        pltpu.VMEM((K, C), jnp.int32),
        tuple(pltpu.VMEM((C, DH), jnp.float32) for _ in range(NBUF)),
        pltpu.VMEM((C,), jnp.float32),
        pltpu.VMEM((RPT,), jnp.float32),
        pltpu.VMEM_SHARED((NP, DH), jnp.float32),
        pltpu.VMEM_SHARED((NP,), jnp.float32),
        tuple(pltpu.SemaphoreType.DMA for _ in range(NBUF)),
    )

    def body(x_hbm, src_hbm, dst_hbm, agg_out, *rest):
        if with_cnt:
            cnt_out = rest[0]
            rest = rest[1:]
        else:
            cnt_out = None
        src_v, dst_v, rows, ones_v, cntbuf, acc_sh, cnt_sh, gsems = rest
        rows0 = rows[0]
        cid = lax.axis_index("c")
        sid = lax.axis_index("s")

        z16 = jnp.zeros((16,), jnp.float32)

        def zrow(i, _):
            for k in range(DH // 16):
                rows0[i, pl.ds(k * 16, 16)] = z16
            return 0

        lax.fori_loop(0, C, zrow, 0)

        def zcnt(i, _):
            cntbuf[pl.ds(i * 16, 16)] = z16
            return 0

        lax.fori_loop(0, RPT // 16, zcnt, 0)
        for k in range(C // 16):
            ones_v[pl.ds(k * 16, 16)] = jnp.ones((16,), jnp.float32)

        base = pl.multiple_of(sid * RPT, RPT)
        for t in range(RPT // C):
            pltpu.sync_copy(rows0, acc_sh.at[pl.ds(base + t * C, C)])
        if with_cnt:
            pltpu.sync_copy(cntbuf, cnt_sh.at[pl.ds(base, RPT)])
        plsc.subcore_barrier()

        pltpu.sync_copy(src_hbm.at[sid], src_v)
        pltpu.sync_copy(dst_hbm.at[sid], dst_v)

        def gsrc(j):
            return x_hbm.at[cid].at[src_v.at[j]]

        def gather(j, b):
            return pltpu.async_copy(gsrc(j), rows[b], gsems[b])

        def gwait(j, b):
            pltpu.make_async_copy(gsrc(j), rows[b], gsems[b]).wait()

        def scat(j, b):
            pltpu.sync_copy(rows[b], acc_sh.at[dst_v.at[j]], add=True)
            if with_cnt:
                do_cnt = jnp.where(cid == 0, j < K // 2, j >= K // 2)

                @pl.when(do_cnt)
                def _():
                    pltpu.sync_copy(ones_v, cnt_sh.at[dst_v.at[j]], add=True)

        for b in range(NBUF):
            gather(b, b)

        def step(g, _):
            j = NBUF * g
            for b in range(NBUF):
                gwait(j + b, b)
                scat(j + b, b)
                gather(j + b + NBUF, b)
            return 0

        lax.fori_loop(0, K // NBUF - 1, step, 0)
        for b in range(NBUF):
            j = K - NBUF + b
            gwait(j, b)
            scat(j, b)

        plsc.subcore_barrier()
        for t in range(RPT // C):
            pltpu.sync_copy(acc_sh.at[pl.ds(base + t * C, C)], rows0)
            pltpu.sync_copy(rows0, agg_out.at[cid, pl.ds(base + t * C, C)])
        if with_cnt:
            pltpu.sync_copy(cnt_sh.at[pl.ds(base, RPT)], cntbuf)
            pltpu.sync_copy(cntbuf, cnt_out.at[cid, pl.ds(base, RPT)])

    return pl.kernel(body, out_type=out_type, mesh=mesh,
                     scratch_types=scratch,
                     compiler_params=pltpu.CompilerParams(
                         use_tc_tiling_on_sc=False))


_sc_agg_cnt = _sc_agg_build(with_cnt=True)
_sc_agg = _sc_agg_build(with_cnt=False)


_RB = 1000
_dn = (((1,), (1,)), ((), ()))


def _tc_root_body(x_ref, wr_ref, b_ref, r_ref):
    r_ref[...] = lax.dot_general(
        x_ref[...], wr_ref[...], _dn, preferred_element_type=jnp.float32
    ) + b_ref[...]


_tc_root = pl.pallas_call(
    _tc_root_body,
    grid=(N // _RB,),
    in_specs=[
        pl.BlockSpec((_RB, D), lambda i: (i, 0)),
        pl.BlockSpec((D, D), lambda i: (0, 0)),
        pl.BlockSpec((1, D), lambda i: (0, 0)),
    ],
    out_specs=pl.BlockSpec((_RB, D), lambda i: (i, 0)),
    out_shape=jax.ShapeDtypeStruct((N, D), jnp.float32),
)


def _tc_combine_build(emit_split: bool):
    def bodyfn(agg_ref, cnt_ref, r_ref, wl_ref, out_ref, *split_refs):
        inv = 1.0 / jnp.maximum(cnt_ref[...], 1.0)
        mean = jnp.concatenate([agg_ref[0], agg_ref[1]], axis=1) * inv
        out = lax.dot_general(
            mean, wl_ref[...], _dn, preferred_element_type=jnp.float32
        ) + r_ref[...]
        out_ref[...] = out
        if emit_split:
            split_refs[0][0] = out[:, :DH]
            split_refs[0][1] = out[:, DH:]

    out_specs = [pl.BlockSpec((_RB, D), lambda i: (i, 0))]
    out_shape = [jax.ShapeDtypeStruct((N, D), jnp.float32)]
    if emit_split:
        out_specs.append(pl.BlockSpec((NC, _RB, DH), lambda i: (0, i, 0)))
        out_shape.append(jax.ShapeDtypeStruct((NC, N, DH), jnp.float32))

    return pl.pallas_call(
        bodyfn,
        grid=(N // _RB,),
        in_specs=[
            pl.BlockSpec((NC, _RB, DH), lambda i: (0, i, 0)),
            pl.BlockSpec((_RB, 1), lambda i: (i, 0)),
            pl.BlockSpec((_RB, D), lambda i: (i, 0)),
            pl.BlockSpec((D, D), lambda i: (0, 0)),
        ],
        out_specs=out_specs,
        out_shape=out_shape,
    )


_tc_combine_split = _tc_combine_build(emit_split=True)
_tc_combine = _tc_combine_build(emit_split=False)


def kernel(x, edge_index, x_batch, W1l, b1, W1r, W2l, b2, W2r):
    src = edge_index[0].astype(jnp.int32)
    dst = edge_index[1].astype(jnp.int32)
    npad = E_PAD - E
    src = jnp.concatenate([src, jnp.arange(npad, dtype=jnp.int32) % N])
    dst = jnp.concatenate([dst, N + (jnp.arange(npad, dtype=jnp.int32) % (NP - N))])
    src_r = src.reshape(NS, K, C)
    dst_r = dst.reshape(NS, K, C)
    x_split = x.reshape(N, NC, DH).transpose(1, 0, 2)

    agg1, cnt = _sc_agg_cnt(x_split, src_r, dst_r)
    cnt_col = (cnt[0, :N] + cnt[1, :N]).reshape(N, 1)
    r1 = _tc_root(x, W1r, b1.reshape(1, D))
    h, h_split = _tc_combine_split(agg1, cnt_col, r1, W1l)
    agg2 = _sc_agg(h_split, src_r, dst_r)
    r2 = _tc_root(h, W2r, b2.reshape(1, D))
    (out,) = _tc_combine(agg2, cnt_col, r2, W2l)
    return out


# speedup vs baseline: 1.3994x; 1.1215x over previous
import functools

import jax
import jax.numpy as jnp
from jax import lax
from jax.experimental import pallas as pl
from jax.experimental.pallas import tpu as pltpu
from jax.experimental.pallas import tpu_sc as plsc

N = 10000
D = 128
DH = D // 2
E = 320000

NC = 2
NS = 16
C = 128
K = 160
NBUF = 4
EPT = K * C
E_PAD = NS * EPT
NP = 10240
RPT = NP // NS


def _sc_agg_build(with_cnt: bool):
    mesh = plsc.VectorSubcoreMesh(core_axis_name="c", subcore_axis_name="s")
    if with_cnt:
        out_type = (jax.ShapeDtypeStruct((NC, NP, DH), jnp.float32),
                    jax.ShapeDtypeStruct((NC, NP), jnp.float32))
    else:
        out_type = jax.ShapeDtypeStruct((NC, NP, DH), jnp.float32)

    scratch = (
        pltpu.VMEM((K, C), jnp.int32),
        pltpu.VMEM((K, C), jnp.int32),
        tuple(pltpu.VMEM((C, DH), jnp.float32) for _ in range(NBUF)),
        pltpu.VMEM((C,), jnp.float32),
        pltpu.VMEM((RPT,), jnp.float32),
        pltpu.VMEM_SHARED((NP, DH), jnp.float32),
        pltpu.VMEM_SHARED((NP,), jnp.float32),
        tuple(pltpu.SemaphoreType.DMA for _ in range(NBUF)),
    )

    def body(x_hbm, src_hbm, dst_hbm, agg_out, *rest):
        if with_cnt:
            cnt_out = rest[0]
            rest = rest[1:]
        else:
            cnt_out = None
        src_v, dst_v, rows, ones_v, cntbuf, acc_sh, cnt_sh, gsems = rest
        rows0 = rows[0]
        cid = lax.axis_index("c")
        sid = lax.axis_index("s")

        z16 = jnp.zeros((16,), jnp.float32)

        def zrow(i, _):
            for k in range(DH // 16):
                rows0[i, pl.ds(k * 16, 16)] = z16
            return 0

        lax.fori_loop(0, C, zrow, 0)

        def zcnt(i, _):
            cntbuf[pl.ds(i * 16, 16)] = z16
            return 0

        lax.fori_loop(0, RPT // 16, zcnt, 0)
        for k in range(C // 16):
            ones_v[pl.ds(k * 16, 16)] = jnp.ones((16,), jnp.float32)

        base = pl.multiple_of(sid * RPT, RPT)
        for t in range(RPT // C):
            pltpu.sync_copy(rows0, acc_sh.at[pl.ds(base + t * C, C)])
        if with_cnt:
            pltpu.sync_copy(cntbuf, cnt_sh.at[pl.ds(base, RPT)])
        plsc.subcore_barrier()

        pltpu.sync_copy(src_hbm.at[sid], src_v)
        pltpu.sync_copy(dst_hbm.at[sid], dst_v)

        def remap(r, _):
            for k in range(C // 16):
                v = src_v[r, pl.ds(k * 16, 16)]
                src_v[r, pl.ds(k * 16, 16)] = v * NC + cid
            return 0

        lax.fori_loop(0, K, remap, 0)

        def gsrc(j):
            return x_hbm.at[src_v.at[j]]

        def gather(j, b):
            return pltpu.async_copy(gsrc(j), rows[b], gsems[b])

        def gwait(j, b):
            pltpu.make_async_copy(gsrc(j), rows[b], gsems[b]).wait()

        def scat(j, b):
            pltpu.sync_copy(rows[b], acc_sh.at[dst_v.at[j]], add=True)
            if with_cnt:
                do_cnt = jnp.where(cid == 0, j < K // 2, j >= K // 2)

                @pl.when(do_cnt)
                def _():
                    pltpu.sync_copy(ones_v, cnt_sh.at[dst_v.at[j]], add=True)

        for b in range(NBUF):
            gather(b, b)

        def step(g, _):
            j = NBUF * g
            for b in range(NBUF):
                gwait(j + b, b)
                scat(j + b, b)
                gather(j + b + NBUF, b)
            return 0

        lax.fori_loop(0, K // NBUF - 1, step, 0)
        for b in range(NBUF):
            j = K - NBUF + b
            gwait(j, b)
            scat(j, b)

        plsc.subcore_barrier()
        for t in range(RPT // C):
            pltpu.sync_copy(acc_sh.at[pl.ds(base + t * C, C)], rows0)
            pltpu.sync_copy(rows0, agg_out.at[cid, pl.ds(base + t * C, C)])
        if with_cnt:
            pltpu.sync_copy(cnt_sh.at[pl.ds(base, RPT)], cntbuf)
            pltpu.sync_copy(cntbuf, cnt_out.at[cid, pl.ds(base, RPT)])

    return pl.kernel(body, out_type=out_type, mesh=mesh,
                     scratch_types=scratch,
                     compiler_params=pltpu.CompilerParams(
                         use_tc_tiling_on_sc=False))


_sc_agg_cnt = _sc_agg_build(with_cnt=True)
_sc_agg = _sc_agg_build(with_cnt=False)


_RB = 1000
_dn = (((1,), (1,)), ((), ()))


def _tc_root_body(x_ref, wr_ref, b_ref, r_ref):
    r_ref[...] = lax.dot_general(
        x_ref[...], wr_ref[...], _dn, preferred_element_type=jnp.float32
    ) + b_ref[...]


_tc_root = pl.pallas_call(
    _tc_root_body,
    grid=(N // _RB,),
    in_specs=[
        pl.BlockSpec((_RB, D), lambda i: (i, 0)),
        pl.BlockSpec((D, D), lambda i: (0, 0)),
        pl.BlockSpec((1, D), lambda i: (0, 0)),
    ],
    out_specs=pl.BlockSpec((_RB, D), lambda i: (i, 0)),
    out_shape=jax.ShapeDtypeStruct((N, D), jnp.float32),
)


def _tc_combine_build(emit_split: bool):
    def bodyfn(agg_ref, cnt_ref, r_ref, wl_ref, out_ref, *split_refs):
        inv = 1.0 / jnp.maximum(cnt_ref[...], 1.0)
        mean = jnp.concatenate([agg_ref[0], agg_ref[1]], axis=1) * inv
        out = lax.dot_general(
            mean, wl_ref[...], _dn, preferred_element_type=jnp.float32
        ) + r_ref[...]
        out_ref[...] = out
        if emit_split:
            split_refs[0][0] = out[:, :DH]
            split_refs[0][1] = out[:, DH:]

    out_specs = [pl.BlockSpec((_RB, D), lambda i: (i, 0))]
    out_shape = [jax.ShapeDtypeStruct((N, D), jnp.float32)]
    if emit_split:
        out_specs.append(pl.BlockSpec((NC, _RB, DH), lambda i: (0, i, 0)))
        out_shape.append(jax.ShapeDtypeStruct((NC, N, DH), jnp.float32))

    return pl.pallas_call(
        bodyfn,
        grid=(N // _RB,),
        in_specs=[
            pl.BlockSpec((NC, _RB, DH), lambda i: (0, i, 0)),
            pl.BlockSpec((_RB, 1), lambda i: (i, 0)),
            pl.BlockSpec((_RB, D), lambda i: (i, 0)),
            pl.BlockSpec((D, D), lambda i: (0, 0)),
        ],
        out_specs=out_specs,
        out_shape=out_shape,
    )


_tc_combine = _tc_combine_build(emit_split=False)


def kernel(x, edge_index, x_batch, W1l, b1, W1r, W2l, b2, W2r):
    src = edge_index[0].astype(jnp.int32)
    dst = edge_index[1].astype(jnp.int32)
    npad = E_PAD - E
    src = jnp.concatenate([src, jnp.arange(npad, dtype=jnp.int32) % N])
    dst = jnp.concatenate([dst, N + (jnp.arange(npad, dtype=jnp.int32) % (NP - N))])
    src_r = src.reshape(NS, K, C)
    dst_r = dst.reshape(NS, K, C)
    agg1, cnt = _sc_agg_cnt(x.reshape(N * NC, DH), src_r, dst_r)
    cnt_col = (cnt[0, :N] + cnt[1, :N]).reshape(N, 1)
    r1 = _tc_root(x, W1r, b1.reshape(1, D))
    (h,) = _tc_combine(agg1, cnt_col, r1, W1l)
    agg2 = _sc_agg(h.reshape(N * NC, DH), src_r, dst_r)
    r2 = _tc_root(h, W2r, b2.reshape(1, D))
    (out,) = _tc_combine(agg2, cnt_col, r2, W2l)
    return out


# speedup vs baseline: 1.5008x; 1.0725x over previous
import functools

import jax
import jax.numpy as jnp
from jax import lax
from jax.experimental import pallas as pl
from jax.experimental.pallas import tpu as pltpu
from jax.experimental.pallas import tpu_sc as plsc

N = 10000
D = 128
DH = D // 2
E = 320000

NC = 2
NS = 16
C = 128
K = 160
NBUF = 4
EPT = K * C
E_PAD = NS * EPT
NP = 10240
RPT = NP // NS


def _sc_agg_build(with_cnt: bool):
    mesh = plsc.VectorSubcoreMesh(core_axis_name="c", subcore_axis_name="s")
    if with_cnt:
        out_type = (jax.ShapeDtypeStruct((NP, D), jnp.float32),
                    jax.ShapeDtypeStruct((NC, NP), jnp.float32))
    else:
        out_type = jax.ShapeDtypeStruct((NP, D), jnp.float32)

    scratch = (
        pltpu.VMEM((K, C), jnp.int32),
        pltpu.VMEM((K, C), jnp.int32),
        tuple(pltpu.VMEM((C, DH), jnp.float32) for _ in range(NBUF)),
        pltpu.VMEM((C,), jnp.float32),
        pltpu.VMEM((RPT,), jnp.float32),
        pltpu.VMEM_SHARED((NP, DH), jnp.float32),
        pltpu.VMEM_SHARED((NP,), jnp.float32),
        tuple(pltpu.SemaphoreType.DMA for _ in range(NBUF)),
    )

    def body(x_hbm, src_hbm, dst_hbm, agg_out, *rest):
        if with_cnt:
            cnt_out = rest[0]
            rest = rest[1:]
        else:
            cnt_out = None
        src_v, dst_v, rows, ones_v, cntbuf, acc_sh, cnt_sh, gsems = rest
        rows0 = rows[0]
        cid = lax.axis_index("c")
        sid = lax.axis_index("s")

        z16 = jnp.zeros((16,), jnp.float32)

        def zrow(i, _):
            for k in range(DH // 16):
                rows0[i, pl.ds(k * 16, 16)] = z16
            return 0

        lax.fori_loop(0, C, zrow, 0)

        def zcnt(i, _):
            cntbuf[pl.ds(i * 16, 16)] = z16
            return 0

        lax.fori_loop(0, RPT // 16, zcnt, 0)
        for k in range(C // 16):
            ones_v[pl.ds(k * 16, 16)] = jnp.ones((16,), jnp.float32)

        base = pl.multiple_of(sid * RPT, RPT)
        for t in range(RPT // C):
            pltpu.sync_copy(rows0, acc_sh.at[pl.ds(base + t * C, C)])
        if with_cnt:
            pltpu.sync_copy(cntbuf, cnt_sh.at[pl.ds(base, RPT)])
        plsc.subcore_barrier()

        pltpu.sync_copy(src_hbm.at[sid], src_v)
        pltpu.sync_copy(dst_hbm.at[sid], dst_v)

        def remap(r, _):
            for k in range(C // 16):
                v = src_v[r, pl.ds(k * 16, 16)]
                src_v[r, pl.ds(k * 16, 16)] = v * NC + cid
            return 0

        lax.fori_loop(0, K, remap, 0)

        def gsrc(j):
            return x_hbm.at[src_v.at[j]]

        def gather(j, b):
            return pltpu.async_copy(gsrc(j), rows[b], gsems[b])

        def gwait(j, b):
            pltpu.make_async_copy(gsrc(j), rows[b], gsems[b]).wait()

        def scat(j, b):
            pltpu.sync_copy(rows[b], acc_sh.at[dst_v.at[j]], add=True)
            if with_cnt:
                do_cnt = jnp.where(cid == 0, j < K // 2, j >= K // 2)

                @pl.when(do_cnt)
                def _():
                    pltpu.sync_copy(ones_v, cnt_sh.at[dst_v.at[j]], add=True)

        for b in range(NBUF):
            gather(b, b)

        def step(g, _):
            j = NBUF * g
            for b in range(NBUF):
                gwait(j + b, b)
                scat(j + b, b)
                gather(j + b + NBUF, b)
            return 0

        lax.fori_loop(0, K // NBUF - 1, step, 0)
        for b in range(NBUF):
            j = K - NBUF + b
            gwait(j, b)
            scat(j, b)

        plsc.subcore_barrier()
        cbase = pl.multiple_of(cid * DH, DH)
        for t in range(RPT // C):
            pltpu.sync_copy(acc_sh.at[pl.ds(base + t * C, C)], rows0)
            pltpu.sync_copy(rows0, agg_out.at[pl.ds(base + t * C, C),
                                              pl.ds(cbase, DH)])
        if with_cnt:
            pltpu.sync_copy(cnt_sh.at[pl.ds(base, RPT)], cntbuf)
            pltpu.sync_copy(cntbuf, cnt_out.at[cid, pl.ds(base, RPT)])

    return pl.kernel(body, out_type=out_type, mesh=mesh,
                     scratch_types=scratch,
                     compiler_params=pltpu.CompilerParams(
                         use_tc_tiling_on_sc=False))


_sc_agg_cnt = _sc_agg_build(with_cnt=True)
_sc_agg = _sc_agg_build(with_cnt=False)


_RB = 1000
_dn = (((1,), (1,)), ((), ()))


def _tc_root_body(x_ref, wr_ref, b_ref, r_ref):
    r_ref[...] = lax.dot_general(
        x_ref[...], wr_ref[...], _dn, preferred_element_type=jnp.float32
    ) + b_ref[...]


_tc_root = pl.pallas_call(
    _tc_root_body,
    grid=(N // _RB,),
    in_specs=[
        pl.BlockSpec((_RB, D), lambda i: (i, 0)),
        pl.BlockSpec((D, D), lambda i: (0, 0)),
        pl.BlockSpec((1, D), lambda i: (0, 0)),
    ],
    out_specs=pl.BlockSpec((_RB, D), lambda i: (i, 0)),
    out_shape=jax.ShapeDtypeStruct((N, D), jnp.float32),
)


def _tc_combine_build(emit_split: bool):
    def bodyfn(agg_ref, cnt_ref, r_ref, wl_ref, out_ref, *split_refs):
        inv = 1.0 / jnp.maximum(cnt_ref[...], 1.0)
        mean = agg_ref[...] * inv
        out = lax.dot_general(
            mean, wl_ref[...], _dn, preferred_element_type=jnp.float32
        ) + r_ref[...]
        out_ref[...] = out
        if emit_split:
            split_refs[0][0] = out[:, :DH]
            split_refs[0][1] = out[:, DH:]

    out_specs = [pl.BlockSpec((_RB, D), lambda i: (i, 0))]
    out_shape = [jax.ShapeDtypeStruct((N, D), jnp.float32)]
    if emit_split:
        out_specs.append(pl.BlockSpec((NC, _RB, DH), lambda i: (0, i, 0)))
        out_shape.append(jax.ShapeDtypeStruct((NC, N, DH), jnp.float32))

    return pl.pallas_call(
        bodyfn,
        grid=(N // _RB,),
        in_specs=[
            pl.BlockSpec((_RB, D), lambda i: (i, 0)),
            pl.BlockSpec((_RB, 1), lambda i: (i, 0)),
            pl.BlockSpec((_RB, D), lambda i: (i, 0)),
            pl.BlockSpec((D, D), lambda i: (0, 0)),
        ],
        out_specs=out_specs,
        out_shape=out_shape,
    )


_tc_combine = _tc_combine_build(emit_split=False)


def kernel(x, edge_index, x_batch, W1l, b1, W1r, W2l, b2, W2r):
    src = edge_index[0].astype(jnp.int32)
    dst = edge_index[1].astype(jnp.int32)
    npad = E_PAD - E
    src = jnp.concatenate([src, jnp.arange(npad, dtype=jnp.int32) % N])
    dst = jnp.concatenate([dst, N + (jnp.arange(npad, dtype=jnp.int32) % (NP - N))])
    src_r = src.reshape(NS, K, C)
    dst_r = dst.reshape(NS, K, C)
    agg1, cnt = _sc_agg_cnt(x.reshape(N * NC, DH), src_r, dst_r)
    cnt_col = (cnt[0, :N] + cnt[1, :N]).reshape(N, 1)
    r1 = _tc_root(x, W1r, b1.reshape(1, D))
    (h,) = _tc_combine(agg1, cnt_col, r1, W1l)
    agg2 = _sc_agg(h.reshape(N * NC, DH), src_r, dst_r)
    r2 = _tc_root(h, W2r, b2.reshape(1, D))
    (out,) = _tc_combine(agg2, cnt_col, r2, W2l)
    return out
